# baseline (speedup 1.0000x reference)
import jax
import jax.numpy as jnp
from jax import lax
from jax.experimental import pallas as pl
from jax.experimental.pallas import tpu as pltpu
from jax.experimental.pallas import tpu_sc as plsc

B, L = 16384, 50
VOCAB, D = 10000, 32
LP = 57
C = 8
NCG = D // C
NBS = 32 // NCG
RPT = B // NBS
CHUNK = 256
NCHUNK = RPT // CHUNK
TSL = VOCAB * C
TCH = LP * CHUNK
OCH = CHUNK * C


def _sc_body(tok_hbm, tab_hbm, out_hbm, ts, tb, ob, tsem, osem):
    wid = lax.axis_index("s") * 2 + lax.axis_index("c")
    cg = lax.rem(wid, NCG)
    bs = lax.div(wid, NCG)
    lanes = lax.iota(jnp.int32, 16)

    pltpu.sync_copy(tab_hbm.at[pl.ds(cg * TSL, TSL)], ts)
    for c in range(C):
        head = ts[pl.ds(c * VOCAB, 16)]
        ts[pl.ds(c * VOCAB, 16)] = jnp.where(lanes < 1, 0.0, head)

    def fire_tok(k, par):
        pltpu.async_copy(
            tok_hbm.at[pl.ds((bs * NCHUNK + k) * TCH, TCH)], tb[par], tsem[par])

    def out_slice(k):
        return out_hbm.at[pl.ds((cg * NBS * NCHUNK + bs * NCHUNK + k) * OCH, OCH)]

    fire_tok(0, 0)
    fire_tok(1, 1)

    def process_chunk(k, par):
        pltpu.make_async_copy(
            tok_hbm.at[pl.ds((bs * NCHUNK + k) * TCH, TCH)], tb[par],
            tsem[par]).wait()

        @pl.when(k >= 2)
        def _():
            pltpu.make_async_copy(ob[par], out_slice(k - 2), osem[par]).wait()

        def rg_body(rg, _):
            zero = jnp.zeros((16,), jnp.float32)
            one = jnp.ones((16,), jnp.float32)
            tbase = lanes * LP + rg * (16 * LP)

            @plsc.parallel_loop(0, L, unroll=4, carry=tuple([zero] * (C + 1)))
            def tok_body(j, car):
                accs, cnt = car[:C], car[C]
                tok = plsc.load_gather(tb[par], [tbase + j])
                cnt = cnt + jnp.where(tok != 0, one, zero)
                accs = tuple(
                    accs[c] + plsc.load_gather(ts, [tok + c * VOCAB])
                    for c in range(C))
                return accs + (cnt,)

            accs, cnt = tok_body[:C], tok_body[C]
            scale = 1.0 / jnp.maximum(cnt, 1.0)
            base_idx = lanes * C + rg * (16 * C)
            for c in range(C):
                plsc.store_scatter(ob[par], [base_idx + c], accs[c] * scale)
            return 0

        lax.fori_loop(0, CHUNK // 16, rg_body, 0)
        pltpu.async_copy(ob[par], out_slice(k), osem[par])

        @pl.when(k + 2 < NCHUNK)
        def _():
            fire_tok(k + 2, par)

    def chunk_loop(g, _):
        for par in (0, 1):
            process_chunk(2 * g + par, par)
        return 0

    lax.fori_loop(0, NCHUNK // 2, chunk_loop, 0)
    for par in (0, 1):
        pltpu.make_async_copy(ob[par], out_slice(NCHUNK - 2 + par),
                              osem[par]).wait()


@jax.jit
def kernel(tokens, table):
    tab_r = table.reshape(VOCAB, NCG, C).transpose(1, 2, 0).reshape(-1)
    tok_r = jnp.pad(tokens, ((0, 0), (0, LP - L))).reshape(-1)
    mesh = plsc.VectorSubcoreMesh(core_axis_name="c", subcore_axis_name="s")
    f = pl.kernel(
        _sc_body,
        out_type=jax.ShapeDtypeStruct((B * D,), jnp.float32),
        mesh=mesh,
        compiler_params=pltpu.CompilerParams(needs_layout_passes=False),
        scratch_types=[
            pltpu.VMEM((TSL,), jnp.float32),
            [pltpu.VMEM((TCH,), jnp.int32) for _ in range(2)],
            [pltpu.VMEM((OCH,), jnp.float32) for _ in range(2)],
            [pltpu.SemaphoreType.DMA for _ in range(2)],
            [pltpu.SemaphoreType.DMA for _ in range(2)],
        ],
    )
    out_r = f(tok_r, tab_r)
    return (out_r.reshape(NCG, NBS, NCHUNK, CHUNK, C)
            .transpose(1, 2, 3, 0, 4).reshape(B, D))


# speedup vs baseline: 1.0200x; 1.0200x over previous
import jax
import jax.numpy as jnp
from jax import lax
from jax.experimental import pallas as pl
from jax.experimental.pallas import tpu as pltpu
from jax.experimental.pallas import tpu_sc as plsc

B, L = 16384, 50
VOCAB, D = 10000, 32
LP = 57
C = 8
NCG = D // C
NBS = 32 // NCG
RPT = B // NBS
CHUNK = 256
NCHUNK = RPT // CHUNK
TSL = VOCAB * C
TCH = LP * CHUNK
OCH = CHUNK * C


def _sc_body(tok_hbm, tab_hbm, out_hbm, ts, tb, ob, tsem, osem):
    wid = lax.axis_index("s") * 2 + lax.axis_index("c")
    cg = lax.rem(wid, NCG)
    bs = lax.div(wid, NCG)
    lanes = lax.iota(jnp.int32, 16)

    pltpu.sync_copy(tab_hbm.at[pl.ds(cg * TSL, TSL)], ts)
    for c in range(C):
        head = ts[pl.ds(c * VOCAB, 16)]
        ts[pl.ds(c * VOCAB, 16)] = jnp.where(lanes < 1, 0.0, head)

    def fire_tok(k, par):
        pltpu.async_copy(
            tok_hbm.at[pl.ds((bs * NCHUNK + k) * TCH, TCH)], tb[par], tsem[par])

    def out_slice(k):
        return out_hbm.at[pl.ds((cg * NBS * NCHUNK + bs * NCHUNK + k) * OCH, OCH)]

    fire_tok(0, 0)
    fire_tok(1, 1)

    def process_chunk(k, par):
        pltpu.make_async_copy(
            tok_hbm.at[pl.ds((bs * NCHUNK + k) * TCH, TCH)], tb[par],
            tsem[par]).wait()

        @pl.when(k >= 2)
        def _():
            pltpu.make_async_copy(ob[par], out_slice(k - 2), osem[par]).wait()

        def rg_body(rg, _):
            zero = jnp.zeros((16,), jnp.float32)
            one = jnp.ones((16,), jnp.float32)
            tbase = lanes * LP + rg * (16 * LP)

            @plsc.parallel_loop(0, L, unroll=2, carry=tuple([zero] * (C + 1)))
            def tok_body(j, car):
                accs, cnt = car[:C], car[C]
                tok = plsc.load_gather(tb[par], [tbase + j])
                cnt = cnt + jnp.where(tok != 0, one, zero)
                accs = tuple(
                    accs[c]
                    + plsc.load_gather(ts.at[pl.ds(c * VOCAB, VOCAB)], [tok])
                    for c in range(C))
                return accs + (cnt,)

            accs, cnt = tok_body[:C], tok_body[C]
            scale = 1.0 / jnp.maximum(cnt, 1.0)
            base_idx = lanes * C + rg * (16 * C)
            for c in range(C):
                plsc.store_scatter(ob[par], [base_idx + c], accs[c] * scale)
            return 0

        lax.fori_loop(0, CHUNK // 16, rg_body, 0)
        pltpu.async_copy(ob[par], out_slice(k), osem[par])

        @pl.when(k + 2 < NCHUNK)
        def _():
            fire_tok(k + 2, par)

    def chunk_loop(g, _):
        for par in (0, 1):
            process_chunk(2 * g + par, par)
        return 0

    lax.fori_loop(0, NCHUNK // 2, chunk_loop, 0)
    for par in (0, 1):
        pltpu.make_async_copy(ob[par], out_slice(NCHUNK - 2 + par),
                              osem[par]).wait()


@jax.jit
def kernel(tokens, table):
    tab_r = table.reshape(VOCAB, NCG, C).transpose(1, 2, 0).reshape(-1)
    tok_r = jnp.pad(tokens, ((0, 0), (0, LP - L))).reshape(-1)
    mesh = plsc.VectorSubcoreMesh(core_axis_name="c", subcore_axis_name="s")
    f = pl.kernel(
        _sc_body,
        out_type=jax.ShapeDtypeStruct((B * D,), jnp.float32),
        mesh=mesh,
        compiler_params=pltpu.CompilerParams(needs_layout_passes=False),
        scratch_types=[
            pltpu.VMEM((TSL,), jnp.float32),
            [pltpu.VMEM((TCH,), jnp.int32) for _ in range(2)],
            [pltpu.VMEM((OCH,), jnp.float32) for _ in range(2)],
            [pltpu.SemaphoreType.DMA for _ in range(2)],
            [pltpu.SemaphoreType.DMA for _ in range(2)],
        ],
    )
    out_r = f(tok_r, tab_r)
    return (out_r.reshape(NCG, NBS, NCHUNK, CHUNK, C)
            .transpose(1, 2, 3, 0, 4).reshape(B, D))


# speedup vs baseline: 1.0685x; 1.0476x over previous
import jax
import jax.numpy as jnp
from jax import lax
from jax.experimental import pallas as pl
from jax.experimental.pallas import tpu as pltpu
from jax.experimental.pallas import tpu_sc as plsc

B, L = 16384, 50
VOCAB, D = 10000, 32
LP = 50
C = 8
NCG = D // C
NBS = 32 // NCG
RPT = B // NBS
CHUNK = 256
NCHUNK = RPT // CHUNK
TSL = VOCAB * C
TCH = LP * CHUNK
OCH = CHUNK * C


def _sc_body(tok_hbm, tab_hbm, out_hbm, ts, tb, ob, tsem, osem):
    wid = lax.axis_index("s") * 2 + lax.axis_index("c")
    cg = lax.rem(wid, NCG)
    bs = lax.div(wid, NCG)
    lanes = lax.iota(jnp.int32, 16)

    pltpu.sync_copy(tab_hbm.at[pl.ds(cg * TSL, TSL)], ts)
    for c in range(C):
        head = ts[pl.ds(c * VOCAB, 16)]
        ts[pl.ds(c * VOCAB, 16)] = jnp.where(lanes < 1, 0.0, head)

    def fire_tok(k, par):
        pltpu.async_copy(
            tok_hbm.at[pl.ds((bs * NCHUNK + k) * TCH, TCH)], tb[par], tsem[par])

    def out_slice(k):
        return out_hbm.at[pl.ds((cg * NBS * NCHUNK + bs * NCHUNK + k) * OCH, OCH)]

    fire_tok(0, 0)
    fire_tok(1, 1)

    def process_chunk(k, par):
        pltpu.make_async_copy(
            tok_hbm.at[pl.ds((bs * NCHUNK + k) * TCH, TCH)], tb[par],
            tsem[par]).wait()

        @pl.when(k >= 2)
        def _():
            pltpu.make_async_copy(ob[par], out_slice(k - 2), osem[par]).wait()

        def rg_body(rg, _):
            zero = jnp.zeros((16,), jnp.float32)
            one = jnp.ones((16,), jnp.float32)
            tbase = lanes * LP + rg * (16 * LP)

            @plsc.parallel_loop(0, L, unroll=2, carry=tuple([zero] * (C + 1)))
            def tok_body(j, car):
                accs, cnt = car[:C], car[C]
                tok = plsc.load_gather(tb[par], [tbase + j])
                cnt = cnt + jnp.where(tok != 0, one, zero)
                accs = tuple(
                    accs[c]
                    + plsc.load_gather(ts.at[pl.ds(c * VOCAB, VOCAB)], [tok])
                    for c in range(C))
                return accs + (cnt,)

            accs, cnt = tok_body[:C], tok_body[C]
            scale = 1.0 / jnp.maximum(cnt, 1.0)
            base_idx = lanes * C + rg * (16 * C)
            for c in range(C):
                plsc.store_scatter(ob[par], [base_idx + c], accs[c] * scale)
            return 0

        lax.fori_loop(0, CHUNK // 16, rg_body, 0)
        pltpu.async_copy(ob[par], out_slice(k), osem[par])

        @pl.when(k + 2 < NCHUNK)
        def _():
            fire_tok(k + 2, par)

    def chunk_loop(g, _):
        for par in (0, 1):
            process_chunk(2 * g + par, par)
        return 0

    lax.fori_loop(0, NCHUNK // 2, chunk_loop, 0)
    for par in (0, 1):
        pltpu.make_async_copy(ob[par], out_slice(NCHUNK - 2 + par),
                              osem[par]).wait()


@jax.jit
def kernel(tokens, table):
    tab_r = table.reshape(VOCAB, NCG, C).transpose(1, 2, 0).reshape(-1)
    tok_r = tokens.reshape(-1)
    mesh = plsc.VectorSubcoreMesh(core_axis_name="c", subcore_axis_name="s")
    f = pl.kernel(
        _sc_body,
        out_type=jax.ShapeDtypeStruct((B * D,), jnp.float32),
        mesh=mesh,
        compiler_params=pltpu.CompilerParams(needs_layout_passes=False),
        scratch_types=[
            pltpu.VMEM((TSL,), jnp.float32),
            [pltpu.VMEM((TCH,), jnp.int32) for _ in range(2)],
            [pltpu.VMEM((OCH,), jnp.float32) for _ in range(2)],
            [pltpu.SemaphoreType.DMA for _ in range(2)],
            [pltpu.SemaphoreType.DMA for _ in range(2)],
        ],
    )
    out_r = f(tok_r, tab_r)
    return (out_r.reshape(NCG, NBS, NCHUNK, CHUNK, C)
            .transpose(1, 2, 3, 0, 4).reshape(B, D))


# speedup vs baseline: 1.1010x; 1.0304x over previous
import jax
import jax.numpy as jnp
from jax import lax
from jax.experimental import pallas as pl
from jax.experimental.pallas import tpu as pltpu
from jax.experimental.pallas import tpu_sc as plsc

B, L = 16384, 50
BH = B // 2
VOCAB, D = 10000, 32
LP = 50
C = 8
NCG = D // C
NBS = 32 // NCG
RPT = BH // NBS
CHUNK = 256
NCHUNK = RPT // CHUNK
TSL = VOCAB * C
TCH = LP * CHUNK
OCH = CHUNK * C


def _sc_body(tok_hbm, tab_hbm, out_hbm, ts, tb, ob, tsem, osem):
    wid = lax.axis_index("s") * 2 + lax.axis_index("c")
    cg = lax.rem(wid, NCG)
    bs = lax.div(wid, NCG)
    lanes = lax.iota(jnp.int32, 16)

    pltpu.sync_copy(tab_hbm.at[pl.ds(cg * TSL, TSL)], ts)
    for c in range(C):
        head = ts[pl.ds(c * VOCAB, 16)]
        ts[pl.ds(c * VOCAB, 16)] = jnp.where(lanes < 1, 0.0, head)

    def fire_tok(k, par):
        pltpu.async_copy(
            tok_hbm.at[pl.ds((bs * NCHUNK + k) * TCH, TCH)], tb[par], tsem[par])

    def out_slice(k):
        return out_hbm.at[pl.ds((cg * NBS * NCHUNK + bs * NCHUNK + k) * OCH, OCH)]

    fire_tok(0, 0)
    fire_tok(1, 1)

    def process_chunk(k, par):
        pltpu.make_async_copy(
            tok_hbm.at[pl.ds((bs * NCHUNK + k) * TCH, TCH)], tb[par],
            tsem[par]).wait()

        @pl.when(k >= 2)
        def _():
            pltpu.make_async_copy(ob[par], out_slice(k - 2), osem[par]).wait()

        def rg_body(rg, _):
            zero = jnp.zeros((16,), jnp.float32)
            one = jnp.ones((16,), jnp.float32)
            tbase = lanes * LP + rg * (16 * LP)

            @plsc.parallel_loop(0, L, unroll=2, carry=tuple([zero] * (C + 1)))
            def tok_body(j, car):
                accs, cnt = car[:C], car[C]
                tok = plsc.load_gather(tb[par], [tbase + j])
                cnt = cnt + jnp.where(tok != 0, one, zero)
                accs = tuple(
                    accs[c]
                    + plsc.load_gather(ts.at[pl.ds(c * VOCAB, VOCAB)], [tok])
                    for c in range(C))
                return accs + (cnt,)

            accs, cnt = tok_body[:C], tok_body[C]
            scale = 1.0 / jnp.maximum(cnt, 1.0)
            base_idx = lanes * C + rg * (16 * C)
            for c in range(C):
                plsc.store_scatter(ob[par], [base_idx + c], accs[c] * scale)
            return 0

        lax.fori_loop(0, CHUNK // 16, rg_body, 0)
        pltpu.async_copy(ob[par], out_slice(k), osem[par])

        @pl.when(k + 2 < NCHUNK)
        def _():
            fire_tok(k + 2, par)

    def chunk_loop(g, _):
        for par in (0, 1):
            process_chunk(2 * g + par, par)
        return 0

    lax.fori_loop(0, NCHUNK // 2, chunk_loop, 0)
    for par in (0, 1):
        pltpu.make_async_copy(ob[par], out_slice(NCHUNK - 2 + par),
                              osem[par]).wait()


@jax.jit
def kernel(tokens, table):
    tab_r = table.reshape(VOCAB, NCG, C).transpose(1, 2, 0).reshape(-1)
    mesh = plsc.VectorSubcoreMesh(core_axis_name="c", subcore_axis_name="s")
    f = pl.kernel(
        _sc_body,
        out_type=jax.ShapeDtypeStruct((BH * D,), jnp.float32),
        mesh=mesh,
        compiler_params=pltpu.CompilerParams(needs_layout_passes=False),
        scratch_types=[
            pltpu.VMEM((TSL,), jnp.float32),
            [pltpu.VMEM((TCH,), jnp.int32) for _ in range(2)],
            [pltpu.VMEM((OCH,), jnp.float32) for _ in range(2)],
            [pltpu.SemaphoreType.DMA for _ in range(2)],
            [pltpu.SemaphoreType.DMA for _ in range(2)],
        ],
    )
    halves = []
    for h in range(2):
        out_r = f(tokens[h * BH:(h + 1) * BH].reshape(-1), tab_r)
        halves.append(out_r.reshape(NCG, NBS, NCHUNK, CHUNK, C)
                      .transpose(1, 2, 3, 0, 4).reshape(BH, D))
    return jnp.concatenate(halves, axis=0)
